# Initial kernel scaffold; baseline (speedup 1.0000x reference)
import jax
import jax.numpy as jnp
from jax.experimental import pallas as pl


def kernel(agts, agt_ctrs, ctx, ctx_ctrs, Wd1, bd1, Wd2, gnd_w, gnd_b, Wq, gnq_w, gnq_b, Wc1, gnc1_w, gnc1_b, Wc2, Wa, norm_w, norm_b, Wl, gnl_w, gnl_b, agt_idcs, ctx_idcs, dist_th):
    raise NotImplementedError("write your pallas kernel here")


# speedup vs baseline: 3.1955x; 3.1955x over previous
import functools

import jax
import jax.numpy as jnp
from jax.experimental import pallas as pl
from jax.experimental.pallas import tpu as pltpu

N_AGT, N_CTX, D, N_C = 10000, 150, 128, 2
A_TILE = 1024
N_PAD = 10240


def _gn(x, w, b, eps=1e-5):
    m = jnp.mean(x, axis=-1, keepdims=True)
    v = jnp.mean((x - m) ** 2, axis=-1, keepdims=True)
    return (x - m) * jax.lax.rsqrt(v + eps) * w + b


def _att_kernel(th_ref, agts_ref, actr_ref, cctr_ref, ctx_ref,
                WqT_ref, WaT_ref, Wd1T_ref, bd1_ref, Wd2T_ref, gnd_w_ref, gnd_b_ref,
                gnq_w_ref, gnq_b_ref, Wc1T_ref, gnc1_w_ref, gnc1_b_ref, Wc2T_ref,
                norm_w_ref, norm_b_ref, WlT_ref, gnl_w_ref, gnl_b_ref,
                out_ref, xc_ref):
    a = agts_ref[:]
    actr = actr_ref[:]
    th = th_ref[0, 0]

    W1T = Wc1T_ref[:]
    w1hT = W1T[0:D, :]
    w1qT = W1T[D:2 * D, :]
    w1xT = W1T[2 * D:3 * D, :]

    dot = functools.partial(jnp.dot, preferred_element_type=jnp.float32)

    q = jax.nn.relu(_gn(dot(a, WqT_ref[:]), gnq_w_ref[:], gnq_b_ref[:]))
    qc = dot(q, w1qT)
    xc_ref[:] = dot(ctx_ref[:], w1xT)

    acc0 = dot(a, WaT_ref[:])

    ax = actr[:, 0:1]
    ay = actr[:, 1:2]
    wd1x = Wd1T_ref[0:1, :]
    wd1y = Wd1T_ref[1:2, :]
    bd1 = bd1_ref[:]

    Wd2T = Wd2T_ref[:]
    Wc2T = Wc2T_ref[:]
    gnd_w, gnd_b = gnd_w_ref[:], gnd_b_ref[:]
    gnc1_w, gnc1_b = gnc1_w_ref[:], gnc1_b_ref[:]

    def body(c, acc):
        cxy = cctr_ref[pl.ds(c, 1), :]
        dx = ax - cxy[:, 0:1]
        dy = ay - cxy[:, 1:2]
        m = jnp.sqrt(dx * dx + dy * dy) <= th
        h1 = jax.nn.relu(dx * wd1x + dy * wd1y + bd1)
        h2 = jax.nn.relu(_gn(dot(h1, Wd2T), gnd_w, gnd_b))
        s = dot(h2, w1hT) + qc + xc_ref[pl.ds(c, 1), :]
        e = dot(jax.nn.relu(_gn(s, gnc1_w, gnc1_b)), Wc2T)
        return acc + jnp.where(m, e, 0.0)

    acc = jax.lax.fori_loop(0, N_CTX, body, acc0)

    o = jax.nn.relu(_gn(acc, norm_w_ref[:], norm_b_ref[:]))
    o = _gn(dot(o, WlT_ref[:]), gnl_w_ref[:], gnl_b_ref[:])
    out_ref[:] = jax.nn.relu(o + a)


def kernel(agts, agt_ctrs, ctx, ctx_ctrs, Wd1, bd1, Wd2, gnd_w, gnd_b, Wq,
           gnq_w, gnq_b, Wc1, gnc1_w, gnc1_b, Wc2, Wa, norm_w, norm_b, Wl,
           gnl_w, gnl_b, agt_idcs, ctx_idcs, dist_th):
    agts_p = jnp.pad(agts, ((0, N_PAD - N_AGT), (0, 0)))
    actr_p = jnp.pad(agt_ctrs, ((0, N_PAD - N_AGT), (0, 0)))
    th = jnp.asarray(dist_th, jnp.float32).reshape(1, 1)

    row = lambda v: v.reshape(1, D)
    n_tiles = N_PAD // A_TILE

    tileA = pl.BlockSpec((A_TILE, D), lambda i: (i, 0))
    tileC = pl.BlockSpec((A_TILE, N_C), lambda i: (i, 0))
    full = lambda s: pl.BlockSpec(s, lambda i: (0,) * len(s))

    out = pl.pallas_call(
        _att_kernel,
        grid=(n_tiles,),
        in_specs=[
            pl.BlockSpec(memory_space=pltpu.SMEM),
            tileA,
            tileC,
            full((N_CTX, N_C)),
            full((N_CTX, D)),
            full((D, D)),
            full((D, D)),
            full((N_C, D)),
            full((1, D)),
            full((D, D)),
            full((1, D)), full((1, D)),
            full((1, D)), full((1, D)),
            full((3 * D, D)),
            full((1, D)), full((1, D)),
            full((D, D)),
            full((1, D)), full((1, D)),
            full((D, D)),
            full((1, D)), full((1, D)),
        ],
        out_specs=tileA,
        out_shape=jax.ShapeDtypeStruct((N_PAD, D), jnp.float32),
        scratch_shapes=[pltpu.VMEM((N_CTX, D), jnp.float32)],
        compiler_params=pltpu.CompilerParams(
            dimension_semantics=("arbitrary",),
        ),
    )(th, agts_p, actr_p, ctx_ctrs, ctx,
      Wq.T, Wa.T, Wd1.T, row(bd1), Wd2.T, row(gnd_w), row(gnd_b),
      row(gnq_w), row(gnq_b), Wc1.T, row(gnc1_w), row(gnc1_b), Wc2.T,
      row(norm_w), row(norm_b), Wl.T, row(gnl_w), row(gnl_b))
    return out[:N_AGT]
